# baseline (speedup 1.0000x reference)
import functools

import jax
import jax.numpy as jnp
from jax import lax
from jax.experimental import pallas as pl
from jax.experimental.pallas import tpu as pltpu
from jax.experimental.pallas import tpu_sc as plsc

N = 10000
DEG = 32
D = 128
UNITS = 128
OUT = 3

NC = 2
NS = 16
NW = NC * NS
RPW = 320
NPAD = NW * RPW
BATCH = 4

_SELU_SCALE = 1.0507009873554805
_SELU_ALPHA = 1.6732632423543772


def _matmul(x, w, bt):
    x = x.astype(jnp.bfloat16)
    w = w.astype(jnp.bfloat16)
    m, k = x.shape
    nrow = m // bt

    def mm(x_ref, w_ref, o_ref):
        o_ref[...] = jnp.dot(x_ref[...], w_ref[...],
                             preferred_element_type=jnp.float32)

    return pl.pallas_call(
        mm,
        grid=(nrow, DEG),
        in_specs=[
            pl.BlockSpec((bt, k), lambda i, kk: (i, 0)),
            pl.BlockSpec((k, UNITS), lambda i, kk: (0, kk)),
        ],
        out_specs=pl.BlockSpec((bt, UNITS), lambda i, kk: (kk * nrow + i, 0)),
        out_shape=jax.ShapeDtypeStruct((DEG * m, UNITS), jnp.float32),
    )(x, w)


def _head(h, w3, b3, bt):
    m = h.shape[0]

    def hk(h_ref, w_ref, b_ref, o_ref):
        logits = jnp.dot(h_ref[...], w_ref[...],
                         preferred_element_type=jnp.float32) + b_ref[...]
        mx = jnp.max(logits, axis=-1, keepdims=True)
        e = jnp.exp(logits - mx)
        o_ref[...] = e / jnp.sum(e, axis=-1, keepdims=True)

    return pl.pallas_call(
        hk,
        grid=(m // bt,),
        in_specs=[
            pl.BlockSpec((bt, UNITS), lambda i: (i, 0)),
            pl.BlockSpec((UNITS, OUT), lambda i: (0, 0)),
            pl.BlockSpec((1, OUT), lambda i: (0, 0)),
        ],
        out_specs=pl.BlockSpec((bt, OUT), lambda i: (i, 0)),
        out_shape=jax.ShapeDtypeStruct((m, OUT), jnp.float32),
    )(h, w3, b3)


RING = 2
NB = RPW // BATCH
IDXN = BATCH * DEG


def _make_sc_layer():
    mesh = plsc.VectorSubcoreMesh(core_axis_name="c", subcore_axis_name="s")

    @functools.partial(
        pl.kernel,
        mesh=mesh,
        out_type=jax.ShapeDtypeStruct((NPAD, UNITS), jnp.float32),
        scratch_types=[
            pltpu.VMEM((RPW, DEG), jnp.int32),
            pltpu.VMEM((NB, IDXN), jnp.int32),
            pltpu.VMEM((RING, IDXN, UNITS), jnp.float32),
            pltpu.VMEM((RING, BATCH, UNITS), jnp.float32),
            pltpu.VMEM((UNITS,), jnp.float32),
            pltpu.SemaphoreType.DMA,
            pltpu.SemaphoreType.DMA,
            pltpu.SemaphoreType.DMA,
            pltpu.SemaphoreType.DMA,
        ],
    )
    def sc_layer(z_hbm, g_hbm, b_hbm, out_hbm, g_all, idx_all, rows_v, o_v,
                 b_v, sem_g0, sem_g1, sem_o0, sem_o1):
        sem_g = (sem_g0, sem_g1)
        sem_o = (sem_o0, sem_o1)
        wid = lax.axis_index("s") * NC + lax.axis_index("c")
        base = wid * RPW
        pltpu.sync_copy(b_hbm, b_v)
        pltpu.sync_copy(g_hbm.at[pl.ds(base, RPW)], g_all)
        iot = lax.iota(jnp.int32, 16)

        def idx_body(bi, carry):
            for r in range(BATCH):
                for h in range(DEG // 16):
                    g16 = g_all[bi * BATCH + r, pl.ds(h * 16, 16)]
                    idx_all[bi, pl.ds(r * DEG + h * 16, 16)] = (
                        g16 + (iot + (h * 16)) * NPAD)
            return carry

        lax.fori_loop(0, NB, idx_body, 0)

        for j in range(RING):
            pltpu.async_copy(z_hbm.at[idx_all.at[j]], rows_v.at[j], sem_g[j])

        def body(g, carry):
            for j in range(RING):
                bi = g * RING + j
                row0 = base + bi * BATCH
                pltpu.make_async_copy(
                    z_hbm.at[idx_all.at[j]], rows_v.at[j], sem_g[j]).wait()

                @pl.when(g > 0)
                def _wait_out():
                    pltpu.make_async_copy(
                        o_v.at[j], out_hbm.at[pl.ds(base, BATCH)],
                        sem_o[j]).wait()

                for r in range(BATCH):
                    for c in range(UNITS // 16):
                        acc = b_v[pl.ds(c * 16, 16)]
                        for k in range(DEG):
                            acc = acc + rows_v[j, r * DEG + k,
                                               pl.ds(c * 16, 16)]
                        res = jnp.where(
                            acc > 0.0,
                            _SELU_SCALE * acc,
                            (_SELU_SCALE * _SELU_ALPHA) * (jnp.exp(acc) - 1.0))
                        o_v[j, r, pl.ds(c * 16, 16)] = res
                pltpu.async_copy(
                    o_v.at[j], out_hbm.at[pl.ds(row0, BATCH)], sem_o[j])

                @pl.when(bi + RING < NB)
                def _fire_next():
                    pltpu.async_copy(
                        z_hbm.at[idx_all.at[bi + RING]], rows_v.at[j],
                        sem_g[j])
            return carry

        lax.fori_loop(0, NB // RING, body, 0)
        for j in range(RING):
            pltpu.make_async_copy(
                o_v.at[j], out_hbm.at[pl.ds(base, BATCH)], sem_o[j]).wait()

    return sc_layer


_sc_layer = _make_sc_layer()


def kernel(X, G, W1, b1, W2, b2, W3, b3):
    Xp = jnp.zeros((NPAD, D), jnp.float32).at[:N].set(X)
    Gp = jnp.zeros((NPAD, DEG), jnp.int32).at[:N].set(G)
    W1R = W1.reshape(DEG, D, UNITS).transpose(1, 0, 2).reshape(D, DEG * UNITS)
    W2R = W2.reshape(DEG, UNITS, UNITS).transpose(1, 0, 2).reshape(
        UNITS, DEG * UNITS)

    Z1 = _matmul(Xp, W1R, 512)
    H1 = _sc_layer(Z1, Gp, b1)
    Z2 = _matmul(H1, W2R, 512)
    H2 = _sc_layer(Z2, Gp, b2)
    P = _head(H2, W3, b3.reshape(1, OUT), 1024)
    return P[:N - 1]


# speedup vs baseline: 2.1354x; 2.1354x over previous
import functools

import jax
import jax.numpy as jnp
from jax import lax
from jax.experimental import pallas as pl
from jax.experimental.pallas import tpu as pltpu
from jax.experimental.pallas import tpu_sc as plsc

N = 10000
DEG = 32
D = 128
UNITS = 128
OUT = 3

NC = 2
NS = 16
NW = NC * NS
RPW = 320
NPAD = NW * RPW
BATCH = 4

_SELU_SCALE = 1.0507009873554805
_SELU_ALPHA = 1.6732632423543772


def _matmul(x, w, bt):
    x = x.astype(jnp.bfloat16)
    w = w.astype(jnp.bfloat16)
    m, k = x.shape

    def mm(x_ref, w_ref, o_ref):
        o_ref[...] = jnp.dot(x_ref[...], w_ref[...],
                             preferred_element_type=jnp.float32)

    return pl.pallas_call(
        mm,
        grid=(DEG,),
        in_specs=[
            pl.BlockSpec((m, k), lambda kk: (0, 0)),
            pl.BlockSpec((k, UNITS), lambda kk: (0, kk)),
        ],
        out_specs=pl.BlockSpec((m, UNITS), lambda kk: (kk, 0)),
        out_shape=jax.ShapeDtypeStruct((DEG * m, UNITS), jnp.float32),
    )(x, w)


def _head(h, w3, b3, bt):
    m = h.shape[0]

    def hk(h_ref, w_ref, b_ref, o_ref):
        logits = jnp.dot(h_ref[...], w_ref[...],
                         preferred_element_type=jnp.float32) + b_ref[...]
        mx = jnp.max(logits, axis=-1, keepdims=True)
        e = jnp.exp(logits - mx)
        o_ref[...] = e / jnp.sum(e, axis=-1, keepdims=True)

    return pl.pallas_call(
        hk,
        grid=(m // bt,),
        in_specs=[
            pl.BlockSpec((bt, UNITS), lambda i: (i, 0)),
            pl.BlockSpec((UNITS, OUT), lambda i: (0, 0)),
            pl.BlockSpec((1, OUT), lambda i: (0, 0)),
        ],
        out_specs=pl.BlockSpec((bt, OUT), lambda i: (i, 0)),
        out_shape=jax.ShapeDtypeStruct((m, OUT), jnp.float32),
    )(h, w3, b3)


RING = 2
NB = RPW // BATCH
IDXN = BATCH * DEG


def _make_sc_layer():
    mesh = plsc.VectorSubcoreMesh(core_axis_name="c", subcore_axis_name="s")

    @functools.partial(
        pl.kernel,
        mesh=mesh,
        out_type=jax.ShapeDtypeStruct((NPAD, UNITS), jnp.float32),
        scratch_types=[
            pltpu.VMEM((RPW, DEG), jnp.int32),
            pltpu.VMEM((NB, IDXN), jnp.int32),
            pltpu.VMEM((RING, IDXN, UNITS), jnp.float32),
            pltpu.VMEM((RING, BATCH, UNITS), jnp.float32),
            pltpu.VMEM((UNITS,), jnp.float32),
            pltpu.SemaphoreType.DMA,
            pltpu.SemaphoreType.DMA,
            pltpu.SemaphoreType.DMA,
            pltpu.SemaphoreType.DMA,
        ],
    )
    def sc_layer(z_hbm, g_hbm, b_hbm, out_hbm, g_all, idx_all, rows_v, o_v,
                 b_v, sem_g0, sem_g1, sem_o0, sem_o1):
        sem_g = (sem_g0, sem_g1)
        sem_o = (sem_o0, sem_o1)
        wid = lax.axis_index("s") * NC + lax.axis_index("c")
        base = wid * RPW
        pltpu.sync_copy(b_hbm, b_v)
        pltpu.sync_copy(g_hbm.at[pl.ds(base, RPW)], g_all)
        iot = lax.iota(jnp.int32, 16)

        def idx_body(bi, carry):
            for r in range(BATCH):
                for h in range(DEG // 16):
                    g16 = g_all[bi * BATCH + r, pl.ds(h * 16, 16)]
                    idx_all[bi, pl.ds(r * DEG + h * 16, 16)] = (
                        g16 + (iot + (h * 16)) * NPAD)
            return carry

        lax.fori_loop(0, NB, idx_body, 0)

        for j in range(RING):
            pltpu.async_copy(z_hbm.at[idx_all.at[j]], rows_v.at[j], sem_g[j])

        def body(g, carry):
            for j in range(RING):
                bi = g * RING + j
                row0 = base + bi * BATCH
                pltpu.make_async_copy(
                    z_hbm.at[idx_all.at[j]], rows_v.at[j], sem_g[j]).wait()

                @pl.when(g > 0)
                def _wait_out():
                    pltpu.make_async_copy(
                        o_v.at[j], out_hbm.at[pl.ds(base, BATCH)],
                        sem_o[j]).wait()

                for r in range(BATCH):
                    for c in range(UNITS // 16):
                        acc = b_v[pl.ds(c * 16, 16)]
                        for k in range(DEG):
                            acc = acc + rows_v[j, r * DEG + k,
                                               pl.ds(c * 16, 16)]
                        res = jnp.where(
                            acc > 0.0,
                            _SELU_SCALE * acc,
                            (_SELU_SCALE * _SELU_ALPHA) * (jnp.exp(acc) - 1.0))
                        o_v[j, r, pl.ds(c * 16, 16)] = res
                pltpu.async_copy(
                    o_v.at[j], out_hbm.at[pl.ds(row0, BATCH)], sem_o[j])

                @pl.when(bi + RING < NB)
                def _fire_next():
                    pltpu.async_copy(
                        z_hbm.at[idx_all.at[bi + RING]], rows_v.at[j],
                        sem_g[j])
            return carry

        lax.fori_loop(0, NB // RING, body, 0)
        for j in range(RING):
            pltpu.make_async_copy(
                o_v.at[j], out_hbm.at[pl.ds(base, BATCH)], sem_o[j]).wait()

    return sc_layer


_sc_layer = _make_sc_layer()


def kernel(X, G, W1, b1, W2, b2, W3, b3):
    Xp = jnp.zeros((NPAD, D), jnp.float32).at[:N].set(X)
    Gp = jnp.zeros((NPAD, DEG), jnp.int32).at[:N].set(G)
    W1R = W1.reshape(DEG, D, UNITS).transpose(1, 0, 2).reshape(D, DEG * UNITS)
    W2R = W2.reshape(DEG, UNITS, UNITS).transpose(1, 0, 2).reshape(
        UNITS, DEG * UNITS)

    Z1 = _matmul(Xp, W1R, 512)
    H1 = _sc_layer(Z1, Gp, b1)
    Z2 = _matmul(H1, W2R, 512)
    H2 = _sc_layer(Z2, Gp, b2)
    P = _head(H2, W3, b3.reshape(1, OUT), 1024)
    return P[:N - 1]
